# baseline (speedup 1.0000x reference)
import functools

import jax
import jax.numpy as jnp
from jax.experimental import pallas as pl
from jax.experimental.pallas import tpu as pltpu

EMB = 128
NP = 100
NC = 500
N = 50000
TILE = 3072
GRID = -(-N // TILE)
F32 = jnp.float32
BF16 = jnp.bfloat16


def _bdot(a, b):
    return jax.lax.dot_general(a.astype(BF16), b.astype(BF16),
                               (((1,), (0,)), ((), ())),
                               preferred_element_type=F32)


def _rowsum_lanes(x):
    ones = jnp.ones((1, EMB), BF16)
    return jax.lax.dot_general(ones, x.astype(BF16), (((1,), (1,)), ((), ())),
                               preferred_element_type=F32)


def _masks(i):
    rows = jax.lax.broadcasted_iota(jnp.int32, (TILE, 1), 0) + i * TILE
    lanes = jax.lax.broadcasted_iota(jnp.int32, (1, TILE), 1) + i * TILE
    return rows < N, lanes < N


def _layer_body(e_ref, advp_ref, advc_ref, adpv_ref, adcv_ref,
                pe_ref, ce_ref, mpv_ref, mcv_ref,
                ga1_ref, ga2_ref, ga3_ref, g1_ref, g2_ref, gb_ref,
                item_ref, numpv_ref, denpv_ref, zpv_ref,
                numcv_ref, dencv_ref, zcv_ref,
                shpv_sc, shcv_sc):
    i = pl.program_id(0)

    @pl.when(i == 0)
    def _():
        for ref in (numpv_ref, denpv_ref, zpv_ref, numcv_ref, dencv_ref,
                    zcv_ref):
            ref[...] = jnp.zeros(ref.shape, ref.dtype)
        shpv_sc[...] = jnp.full(shpv_sc.shape, -jnp.inf, F32)
        shcv_sc[...] = jnp.full(shcv_sc.shape, -jnp.inf, F32)

    def body(masked):
        if masked:
            rmask, lmask = _masks(i)
            e = jnp.where(rmask, e_ref[...], 0.0)
        else:
            e = e_ref[...]
        evp = _bdot(advp_ref[...], pe_ref[...])
        evc = _bdot(advc_ref[...], ce_ref[...])
        garg = (_bdot(e, ga1_ref[...]) + _bdot(evp, ga2_ref[...])
                + _bdot(evc, ga3_ref[...]) + _bdot(evp, g1_ref[...])
                + _bdot(evc, g2_ref[...]) + gb_ref[...])
        g = jax.nn.sigmoid(garg)
        item = e + g * evp + (1.0 - g) * evc
        item_ref[...] = item

        s = _rowsum_lanes(e)
        if masked:
            smx = jnp.max(jnp.where(lmask, s, -jnp.inf), axis=(0, 1),
                          keepdims=True)
            smn = jnp.min(jnp.where(lmask, s, jnp.inf), axis=(0, 1),
                          keepdims=True)
        else:
            smx = jnp.max(s, axis=(0, 1), keepdims=True)
            smn = jnp.min(s, axis=(0, 1), keepdims=True)

        def intra_part(m, adj, num_ref, den_ref, z_ref, sh_ref):
            sh_t = jnp.where(m >= 0.0, m * smx, m * smn)
            p = jnp.exp(m * s - sh_t)
            if masked:
                p = jnp.where(lmask, p, 0.0)
                w = jnp.where(lmask, adj * p, 0.0)
            else:
                w = adj * p
            sh_old = sh_ref[...]
            sh_new = jnp.maximum(sh_old, sh_t)
            alpha = jnp.exp(sh_old - sh_new)
            beta = jnp.exp(sh_t - sh_new)
            num_ref[...] = num_ref[...] * alpha + _bdot(w, e) * beta
            den_ref[...] = (den_ref[...] * alpha
                            + jnp.sum(w, axis=1, keepdims=True) * beta)
            z_ref[...] = (z_ref[...] * alpha
                          + jnp.sum(p, axis=1, keepdims=True) * beta)
            sh_ref[...] = sh_new

        intra_part(mpv_ref[...], adpv_ref[...],
                   numpv_ref, denpv_ref, zpv_ref, shpv_sc)
        intra_part(mcv_ref[...], adcv_ref[...],
                   numcv_ref, dencv_ref, zcv_ref, shcv_sc)

    @pl.when(i < GRID - 1)
    def _():
        body(masked=False)

    @pl.when(i == GRID - 1)
    def _():
        body(masked=True)


def _layer(e, advp, advc, adpv, adcv, pe, ce, mpv, mcv,
           ga1, ga2, ga3, g1, g2, gb):
    c = lambda *shape: pl.BlockSpec(shape, lambda i: (0,) * len(shape))
    return pl.pallas_call(
        _layer_body,
        grid=(GRID,),
        in_specs=[
            pl.BlockSpec((TILE, EMB), lambda i: (i, 0)),
            pl.BlockSpec((TILE, NP), lambda i: (i, 0)),
            pl.BlockSpec((TILE, NC), lambda i: (i, 0)),
            pl.BlockSpec((NP, TILE), lambda i: (0, i)),
            pl.BlockSpec((NC, TILE), lambda i: (0, i)),
            c(NP, EMB), c(NC, EMB), c(NP, 1), c(NC, 1),
            c(EMB, EMB), c(EMB, EMB), c(EMB, EMB), c(EMB, EMB), c(EMB, EMB),
            c(1, EMB),
        ],
        out_specs=[
            pl.BlockSpec((TILE, EMB), lambda i: (i, 0)),
            c(NP, EMB), c(NP, 1), c(NP, 1),
            c(NC, EMB), c(NC, 1), c(NC, 1),
        ],
        out_shape=[
            jax.ShapeDtypeStruct((N, EMB), F32),
            jax.ShapeDtypeStruct((NP, EMB), F32),
            jax.ShapeDtypeStruct((NP, 1), F32),
            jax.ShapeDtypeStruct((NP, 1), F32),
            jax.ShapeDtypeStruct((NC, EMB), F32),
            jax.ShapeDtypeStruct((NC, 1), F32),
            jax.ShapeDtypeStruct((NC, 1), F32),
        ],
        scratch_shapes=[pltpu.VMEM((NP, 1), F32), pltpu.VMEM((NC, 1), F32)],
    )(e, advp, advc, adpv, adcv, pe, ce, mpv, mcv,
      ga1, ga2, ga3, g1, g2, gb)


def _finalize_body(pe_ref, ce_ref, numpv_ref, denpv_ref, zpv_ref,
                   numcv_ref, dencv_ref, zcv_ref, adpc_ref, adcp_ref,
                   mpc_ref, mcp_ref,
                   pa1_ref, pa2_ref, pa3_ref, p1_ref, p2_ref, pb_ref,
                   ca1_ref, ca2_ref, ca3_ref, c1_ref, c2_ref, cb_ref,
                   price_ref, cate_ref):
    pe = pe_ref[...]
    ce = ce_ref[...]
    e_pv = numpv_ref[...] / (denpv_ref[...] + 1e-8 * zpv_ref[...])
    e_cv = numcv_ref[...] / (dencv_ref[...] + 1e-8 * zcv_ref[...])

    def intra_small(adj, m, emb2):
        s = _rowsum_lanes(emb2)
        mn = jnp.min(s, axis=(0, 1), keepdims=True)
        mx = jnp.max(s, axis=(0, 1), keepdims=True)
        shift = jnp.where(m >= 0.0, m * mx, m * mn)
        p = jnp.exp(m * s - shift)
        w = adj * p
        num = _bdot(w, emb2)
        den = jnp.sum(w, axis=1, keepdims=True)
        z = jnp.sum(p, axis=1, keepdims=True)
        return num / (den + 1e-8 * z)

    e_pc = intra_small(adpc_ref[...], mpc_ref[...], ce)
    e_cp = intra_small(adcp_ref[...], mcp_ref[...], pe)

    gp = jax.nn.sigmoid(_bdot(pe, pa1_ref[...]) + _bdot(e_pv, pa2_ref[...])
                        + _bdot(e_pc, pa3_ref[...]) + _bdot(e_pv, p1_ref[...])
                        + _bdot(e_pc, p2_ref[...]) + pb_ref[...])
    price_ref[...] = pe + gp * e_pv + (1.0 - gp) * e_pc
    gc = jax.nn.sigmoid(_bdot(ce, ca1_ref[...]) + _bdot(e_cp, ca2_ref[...])
                        + _bdot(e_cv, ca3_ref[...]) + _bdot(e_cp, c1_ref[...])
                        + _bdot(e_cv, c2_ref[...]) + cb_ref[...])
    cate_ref[...] = ce + gc * e_cp + (1.0 - gc) * e_cv


def _finalize(pe, ce, numpv, denpv, zpv, numcv, dencv, zcv, adpc, adcp,
              mpc, mcp, pw, cw):
    return pl.pallas_call(
        _finalize_body,
        out_shape=[jax.ShapeDtypeStruct((NP, EMB), F32),
                   jax.ShapeDtypeStruct((NC, EMB), F32)],
    )(pe, ce, numpv, denpv, zpv, numcv, dencv, zcv, adpc, adcp,
      mpc, mcp, *pw, *cw)


def kernel(adjacency, adjacency_pv, adjacency_vp, adjacency_pc, adjacency_cp,
           adjacency_cv, adjacency_vc, embedding, pri_emb, cate_emb,
           single_basket, session_basket, mat_pv, mat_pc, mat_cp, mat_cv,
           W_aogi, b_aogi, W_bgi1, b_bgi1, W_bgi2, b_bgi2,
           W_aogp, b_aogp, W_bgp1, b_bgp1, W_bgp2, b_bgp2,
           W_aogc, b_aogc, W_bgc1, b_bgc1, W_bgc2, b_bgc2):
    def split(Wa, ba, W1, b1, W2, b2):
        return (Wa[:EMB], Wa[EMB:2 * EMB], Wa[2 * EMB:], W1, W2,
                (ba + b1 + b2)[None, :])

    gi = split(W_aogi, b_aogi, W_bgi1, b_bgi1, W_bgi2, b_bgi2)
    gp = split(W_aogp, b_aogp, W_bgp1, b_bgp1, W_bgp2, b_bgp2)
    gc = split(W_aogc, b_aogc, W_bgc1, b_bgc1, W_bgc2, b_bgc2)

    rd = lambda m: m.astype(BF16).astype(F32)
    mpv, mpc, mcp, mcv = rd(mat_pv), rd(mat_pc), rd(mat_cp), rd(mat_cv)

    e, pe, ce = embedding, pri_emb, cate_emb
    for layer in range(2):
        (item, numpv, denpv, zpv, numcv, dencv, zcv) = _layer(
            e, adjacency_vp, adjacency_vc, adjacency_pv, adjacency_cv,
            pe, ce, mpv, mcv, *gi)
        price, cate = _finalize(pe, ce, numpv, denpv, zpv, numcv, dencv, zcv,
                                adjacency_pc, adjacency_cp, mpc, mcp, gp, gc)
        e, pe, ce = item, price, cate
    return (e, pe, ce)


# speedup vs baseline: 1.1428x; 1.1428x over previous
import functools

import jax
import jax.numpy as jnp
from jax.experimental import pallas as pl
from jax.experimental.pallas import tpu as pltpu

EMB = 128
NP = 100
NC = 500
N = 50000
TILE = 2048
GRID = -(-N // TILE)
F32 = jnp.float32
BF16 = jnp.bfloat16


def _bdot(a, b):
    return jax.lax.dot_general(a.astype(BF16), b.astype(BF16),
                               (((1,), (0,)), ((), ())),
                               preferred_element_type=F32)


def _rowsum_lanes(x):
    ones = jnp.ones((1, EMB), BF16)
    return jax.lax.dot_general(ones, x.astype(BF16), (((1,), (1,)), ((), ())),
                               preferred_element_type=F32)


def _masks(i):
    rows = jax.lax.broadcasted_iota(jnp.int32, (TILE, 1), 0) + i * TILE
    lanes = jax.lax.broadcasted_iota(jnp.int32, (1, TILE), 1) + i * TILE
    return rows < N, lanes < N


def _layer_body(e_ref, advp_ref, advc_ref, adpv_ref, adcv_ref,
                pe_ref, ce_ref, mpv_ref, mcv_ref,
                ga1_ref, ga2_ref, ga3_ref, g1_ref, g2_ref, gb_ref,
                item_ref, numpv_ref, denpv_ref, zpv_ref,
                numcv_ref, dencv_ref, zcv_ref,
                shpv_sc, shcv_sc):
    i = pl.program_id(0)

    @pl.when(i == 0)
    def _():
        for ref in (numpv_ref, denpv_ref, zpv_ref, numcv_ref, dencv_ref,
                    zcv_ref):
            ref[...] = jnp.zeros(ref.shape, ref.dtype)
        shpv_sc[...] = jnp.full(shpv_sc.shape, -jnp.inf, F32)
        shcv_sc[...] = jnp.full(shcv_sc.shape, -jnp.inf, F32)

    def body(masked):
        if masked:
            rmask, lmask = _masks(i)
            e = jnp.where(rmask, e_ref[...], 0.0)
        else:
            e = e_ref[...]
        evp = _bdot(advp_ref[...], pe_ref[...])
        evc = _bdot(advc_ref[...], ce_ref[...])
        garg = (_bdot(e, ga1_ref[...]) + _bdot(evp, ga2_ref[...])
                + _bdot(evc, ga3_ref[...]) + _bdot(evp, g1_ref[...])
                + _bdot(evc, g2_ref[...]) + gb_ref[...])
        g = jax.nn.sigmoid(garg)
        item = e + g * evp + (1.0 - g) * evc
        item_ref[...] = item

        s = _rowsum_lanes(e)
        if masked:
            smx = jnp.max(jnp.where(lmask, s, -jnp.inf), axis=(0, 1),
                          keepdims=True)
            smn = jnp.min(jnp.where(lmask, s, jnp.inf), axis=(0, 1),
                          keepdims=True)
        else:
            smx = jnp.max(s, axis=(0, 1), keepdims=True)
            smn = jnp.min(s, axis=(0, 1), keepdims=True)

        def intra_part(m, adj, num_ref, den_ref, z_ref, sh_ref):
            sh_t = jnp.where(m >= 0.0, m * smx, m * smn)
            p = jnp.exp(m * s - sh_t)
            if masked:
                p = jnp.where(lmask, p, 0.0)
                w = jnp.where(lmask, adj * p, 0.0)
            else:
                w = adj * p
            sh_old = sh_ref[...]
            sh_new = jnp.maximum(sh_old, sh_t)
            alpha = jnp.exp(sh_old - sh_new)
            beta = jnp.exp(sh_t - sh_new)
            num_ref[...] = num_ref[...] * alpha + _bdot(w, e) * beta
            den_ref[...] = (den_ref[...] * alpha
                            + jnp.sum(w, axis=1, keepdims=True) * beta)
            z_ref[...] = (z_ref[...] * alpha
                          + jnp.sum(p, axis=1, keepdims=True) * beta)
            sh_ref[...] = sh_new

        intra_part(mpv_ref[...], adpv_ref[...],
                   numpv_ref, denpv_ref, zpv_ref, shpv_sc)
        intra_part(mcv_ref[...], adcv_ref[...],
                   numcv_ref, dencv_ref, zcv_ref, shcv_sc)

    @pl.when(i < GRID - 1)
    def _():
        body(masked=False)

    @pl.when(i == GRID - 1)
    def _():
        body(masked=True)


def _layer(e, advp, advc, adpv, adcv, pe, ce, mpv, mcv,
           ga1, ga2, ga3, g1, g2, gb):
    c = lambda *shape: pl.BlockSpec(shape, lambda i: (0,) * len(shape))
    return pl.pallas_call(
        _layer_body,
        grid=(GRID,),
        in_specs=[
            pl.BlockSpec((TILE, EMB), lambda i: (i, 0)),
            pl.BlockSpec((TILE, NP), lambda i: (i, 0)),
            pl.BlockSpec((TILE, NC), lambda i: (i, 0)),
            pl.BlockSpec((NP, TILE), lambda i: (0, i)),
            pl.BlockSpec((NC, TILE), lambda i: (0, i)),
            c(NP, EMB), c(NC, EMB), c(NP, 1), c(NC, 1),
            c(EMB, EMB), c(EMB, EMB), c(EMB, EMB), c(EMB, EMB), c(EMB, EMB),
            c(1, EMB),
        ],
        out_specs=[
            pl.BlockSpec((TILE, EMB), lambda i: (i, 0)),
            c(NP, EMB), c(NP, 1), c(NP, 1),
            c(NC, EMB), c(NC, 1), c(NC, 1),
        ],
        out_shape=[
            jax.ShapeDtypeStruct((N, EMB), F32),
            jax.ShapeDtypeStruct((NP, EMB), F32),
            jax.ShapeDtypeStruct((NP, 1), F32),
            jax.ShapeDtypeStruct((NP, 1), F32),
            jax.ShapeDtypeStruct((NC, EMB), F32),
            jax.ShapeDtypeStruct((NC, 1), F32),
            jax.ShapeDtypeStruct((NC, 1), F32),
        ],
        scratch_shapes=[pltpu.VMEM((NP, 1), F32), pltpu.VMEM((NC, 1), F32)],
    )(e, advp, advc, adpv, adcv, pe, ce, mpv, mcv,
      ga1, ga2, ga3, g1, g2, gb)


def _finalize_body(pe_ref, ce_ref, numpv_ref, denpv_ref, zpv_ref,
                   numcv_ref, dencv_ref, zcv_ref, adpc_ref, adcp_ref,
                   mpc_ref, mcp_ref,
                   pa1_ref, pa2_ref, pa3_ref, p1_ref, p2_ref, pb_ref,
                   ca1_ref, ca2_ref, ca3_ref, c1_ref, c2_ref, cb_ref,
                   price_ref, cate_ref):
    pe = pe_ref[...]
    ce = ce_ref[...]
    e_pv = numpv_ref[...] / (denpv_ref[...] + 1e-8 * zpv_ref[...])
    e_cv = numcv_ref[...] / (dencv_ref[...] + 1e-8 * zcv_ref[...])

    def intra_small(adj, m, emb2):
        s = _rowsum_lanes(emb2)
        mn = jnp.min(s, axis=(0, 1), keepdims=True)
        mx = jnp.max(s, axis=(0, 1), keepdims=True)
        shift = jnp.where(m >= 0.0, m * mx, m * mn)
        p = jnp.exp(m * s - shift)
        w = adj * p
        num = _bdot(w, emb2)
        den = jnp.sum(w, axis=1, keepdims=True)
        z = jnp.sum(p, axis=1, keepdims=True)
        return num / (den + 1e-8 * z)

    e_pc = intra_small(adpc_ref[...], mpc_ref[...], ce)
    e_cp = intra_small(adcp_ref[...], mcp_ref[...], pe)

    gp = jax.nn.sigmoid(_bdot(pe, pa1_ref[...]) + _bdot(e_pv, pa2_ref[...])
                        + _bdot(e_pc, pa3_ref[...]) + _bdot(e_pv, p1_ref[...])
                        + _bdot(e_pc, p2_ref[...]) + pb_ref[...])
    price_ref[...] = pe + gp * e_pv + (1.0 - gp) * e_pc
    gc = jax.nn.sigmoid(_bdot(ce, ca1_ref[...]) + _bdot(e_cp, ca2_ref[...])
                        + _bdot(e_cv, ca3_ref[...]) + _bdot(e_cp, c1_ref[...])
                        + _bdot(e_cv, c2_ref[...]) + cb_ref[...])
    cate_ref[...] = ce + gc * e_cp + (1.0 - gc) * e_cv


def _finalize(pe, ce, numpv, denpv, zpv, numcv, dencv, zcv, adpc, adcp,
              mpc, mcp, pw, cw):
    return pl.pallas_call(
        _finalize_body,
        out_shape=[jax.ShapeDtypeStruct((NP, EMB), F32),
                   jax.ShapeDtypeStruct((NC, EMB), F32)],
    )(pe, ce, numpv, denpv, zpv, numcv, dencv, zcv, adpc, adcp,
      mpc, mcp, *pw, *cw)


def kernel(adjacency, adjacency_pv, adjacency_vp, adjacency_pc, adjacency_cp,
           adjacency_cv, adjacency_vc, embedding, pri_emb, cate_emb,
           single_basket, session_basket, mat_pv, mat_pc, mat_cp, mat_cv,
           W_aogi, b_aogi, W_bgi1, b_bgi1, W_bgi2, b_bgi2,
           W_aogp, b_aogp, W_bgp1, b_bgp1, W_bgp2, b_bgp2,
           W_aogc, b_aogc, W_bgc1, b_bgc1, W_bgc2, b_bgc2):
    def split(Wa, ba, W1, b1, W2, b2):
        return (Wa[:EMB], Wa[EMB:2 * EMB], Wa[2 * EMB:], W1, W2,
                (ba + b1 + b2)[None, :])

    gi = split(W_aogi, b_aogi, W_bgi1, b_bgi1, W_bgi2, b_bgi2)
    gp = split(W_aogp, b_aogp, W_bgp1, b_bgp1, W_bgp2, b_bgp2)
    gc = split(W_aogc, b_aogc, W_bgc1, b_bgc1, W_bgc2, b_bgc2)

    rd = lambda m: m.astype(BF16).astype(F32)
    mpv, mpc, mcp, mcv = rd(mat_pv), rd(mat_pc), rd(mat_cp), rd(mat_cv)

    e, pe, ce = embedding, pri_emb, cate_emb
    for layer in range(2):
        (item, numpv, denpv, zpv, numcv, dencv, zcv) = _layer(
            e, adjacency_vp, adjacency_vc, adjacency_pv, adjacency_cv,
            pe, ce, mpv, mcv, *gi)
        price, cate = _finalize(pe, ce, numpv, denpv, zpv, numcv, dencv, zcv,
                                adjacency_pc, adjacency_cp, mpc, mcp, gp, gc)
        e, pe, ce = item, price, cate
    return (e, pe, ce)
